# baseline (speedup 1.0000x reference)
import jax
import jax.numpy as jnp
from jax import lax
from jax.experimental import pallas as pl


_R = 64


def _tc_body(basis_ref, outcome_ref, table_ref, out_ref):
    r, c = basis_ref.shape
    ids = basis_ref[...] * 2 + outcome_ref[...]
    tab = table_ref[...]
    tok = lax.broadcasted_iota(jnp.int32, (6, c), 0)
    for g in range(r):
        row = jnp.broadcast_to(ids[g:g + 1, :], (6, c))
        onehot = (row == tok).astype(jnp.float32)
        res = lax.dot_general(onehot, tab, (((0,), (0,)), ((), ())),
                              preferred_element_type=jnp.float32)
        out_ref[pl.ds(g * c, c), :] = res


def kernel(basis, outcome, table):
    n, c = basis.shape
    total = n * c

    grid = (n // _R,)
    out = pl.pallas_call(
        _tc_body,
        grid=grid,
        in_specs=[
            pl.BlockSpec((_R, c), lambda i: (i, 0)),
            pl.BlockSpec((_R, c), lambda i: (i, 0)),
            pl.BlockSpec((6, 64), lambda i: (0, 0)),
        ],
        out_specs=pl.BlockSpec((_R * c, 64), lambda i: (i, 0)),
        out_shape=jax.ShapeDtypeStruct((total, 64), jnp.float32),
    )(basis, outcome, table)
    return out.reshape(n, c, 64)
